# Initial kernel scaffold; baseline (speedup 1.0000x reference)
import jax
import jax.numpy as jnp
from jax.experimental import pallas as pl


def kernel(subject_ids):
    raise NotImplementedError("write your pallas kernel here")


# speedup vs baseline: 2.4726x; 2.4726x over previous
import functools

import jax
import jax.numpy as jnp
from jax import lax
from jax.experimental import pallas as pl
from jax.experimental.pallas import tpu as pltpu
from jax.experimental.pallas import tpu_sc as plsc

_N_SUB = 1_000_000
_N_IDS = 16384
_NC = 2
_NS = 16
_NW = _NC * _NS
_CHUNK = 31248
_LAST = _N_SUB - (_NW - 1) * _CHUNK
_SCRATCH = 31360

_mesh = plsc.VectorSubcoreMesh(core_axis_name="c", subcore_axis_name="s")


@functools.partial(
    pl.kernel,
    out_type=jax.ShapeDtypeStruct((_N_SUB,), jnp.float32),
    mesh=_mesh,
    scratch_types=[
        pltpu.VMEM((_N_IDS,), jnp.int32),
        pltpu.VMEM((_SCRATCH,), jnp.float32),
        pltpu.SemaphoreType.DMA,
    ],
    compiler_params=pltpu.CompilerParams(needs_layout_passes=False),
)
def _mask_kernel(ids_hbm, out_hbm, ids_v, chunk_v, sem):
    wid = lax.axis_index("s") * _NC + lax.axis_index("c")
    lo = wid * _CHUNK
    hi = jnp.where(wid == _NW - 1, _N_SUB, lo + _CHUNK)

    ids_copy = pltpu.async_copy(ids_hbm, ids_v, sem)

    zero16 = jnp.zeros((16,), jnp.float32)

    def zero_body(i, carry):
        base = i * 64
        for j in range(4):
            chunk_v[pl.ds(base + j * 16, 16)] = zero16
        return carry

    lax.fori_loop(0, _SCRATCH // 64, zero_body, 0)

    ids_copy.wait()

    ones16 = jnp.full((16,), 1.0, jnp.float32)

    def scatter_body(i, carry):
        base = i * 64
        for j in range(4):
            ids16 = ids_v[pl.ds(base + j * 16, 16)]
            inb = (ids16 >= lo) & (ids16 < hi)
            idx = jnp.where(inb, ids16 - lo, 0)
            plsc.store_scatter(chunk_v, [idx], ones16, mask=inb)
        return carry

    lax.fori_loop(0, _N_IDS // 64, scatter_body, 0)

    @pl.when(wid < _NW - 1)
    def _():
        pltpu.sync_copy(chunk_v.at[pl.ds(0, _CHUNK)], out_hbm.at[pl.ds(lo, _CHUNK)])

    @pl.when(wid == _NW - 1)
    def _():
        pltpu.sync_copy(
            chunk_v.at[pl.ds(0, _LAST)],
            out_hbm.at[pl.ds((_NW - 1) * _CHUNK, _LAST)],
        )


def kernel(subject_ids):
    ids = jnp.reshape(subject_ids, (-1,)).astype(jnp.int32)
    return _mask_kernel(ids)


# speedup vs baseline: 3.0749x; 1.2436x over previous
import functools

import jax
import jax.numpy as jnp
from jax import lax
from jax.experimental import pallas as pl
from jax.experimental.pallas import tpu as pltpu
from jax.experimental.pallas import tpu_sc as plsc

_N_SUB = 1_000_000
_N_IDS = 16384
_NC = 2
_NS = 16
_NW = _NC * _NS
_CHUNK = 31248
_LAST = _N_SUB - (_NW - 1) * _CHUNK
_SCRATCH = 31360

_mesh = plsc.VectorSubcoreMesh(core_axis_name="c", subcore_axis_name="s")


@functools.partial(
    pl.kernel,
    out_type=jax.ShapeDtypeStruct((_N_SUB,), jnp.float32),
    mesh=_mesh,
    scratch_types=[
        pltpu.VMEM((_N_IDS,), jnp.int32),
        pltpu.VMEM((_SCRATCH,), jnp.float32),
        pltpu.SemaphoreType.DMA,
    ],
    compiler_params=pltpu.CompilerParams(needs_layout_passes=False),
)
def _mask_kernel(ids_hbm, out_hbm, ids_v, chunk_v, sem):
    wid = lax.axis_index("s") * _NC + lax.axis_index("c")
    lo = wid * _CHUNK
    hi = jnp.where(wid == _NW - 1, _N_SUB, lo + _CHUNK)

    ids_copy = pltpu.async_copy(ids_hbm, ids_v, sem)

    zero16 = jnp.zeros((16,), jnp.float32)

    @plsc.parallel_loop(0, _SCRATCH, step=128, unroll=2)
    def _(base):
        for j in range(8):
            chunk_v[pl.ds(base + j * 16, 16)] = zero16

    ids_copy.wait()

    ones16 = jnp.full((16,), 1.0, jnp.float32)
    size_u = lax.convert_element_type(hi - lo, jnp.uint32)

    @plsc.parallel_loop(0, _N_IDS, step=128, unroll=2)
    def _(base):
        for j in range(8):
            ids16 = ids_v[pl.ds(base + j * 16, 16)]
            local = ids16 - lo
            inb = plsc.bitcast(local, jnp.uint32) < size_u
            plsc.store_scatter(chunk_v, [local], ones16, mask=inb)

    @pl.when(wid < _NW - 1)
    def _():
        pltpu.sync_copy(chunk_v.at[pl.ds(0, _CHUNK)], out_hbm.at[pl.ds(lo, _CHUNK)])

    @pl.when(wid == _NW - 1)
    def _():
        pltpu.sync_copy(
            chunk_v.at[pl.ds(0, _LAST)],
            out_hbm.at[pl.ds((_NW - 1) * _CHUNK, _LAST)],
        )


def kernel(subject_ids):
    ids = jnp.reshape(subject_ids, (-1,)).astype(jnp.int32)
    return _mask_kernel(ids)
